# baseline (speedup 1.0000x reference)
import jax
import jax.numpy as jnp
from jax.experimental import pallas as pl


def _offsets_body(x_ref, p_ref, o_ref):
    x = x_ref[...]
    o_ref[...] = x - jnp.dot(x, p_ref[...], preferred_element_type=jnp.float32)


@jax.jit
def kernel(positions, parents):
    B, T, J, C = positions.shape
    N = B * T
    D = J * C
    x = positions.reshape(N, D)

    perm = (parents.astype(jnp.int32)[:, None] * C
            + jnp.arange(C, dtype=jnp.int32)[None, :]).reshape(D)
    p = (perm[None, :] == jnp.arange(D, dtype=jnp.int32)[:, None]).astype(
        jnp.float32)

    R = 2048
    out = pl.pallas_call(
        _offsets_body,
        grid=(N // R,),
        in_specs=[
            pl.BlockSpec((R, D), lambda i: (i, 0)),
            pl.BlockSpec((D, D), lambda i: (0, 0)),
        ],
        out_specs=pl.BlockSpec((R, D), lambda i: (i, 0)),
        out_shape=jax.ShapeDtypeStruct((N, D), jnp.float32),
    )(x, p)
    return out.reshape(B, T, J, C)
